# baseline (speedup 1.0000x reference)
import jax
import jax.numpy as jnp
import numpy as np
from jax import lax
from jax.experimental import pallas as pl
from jax.experimental.pallas import tpu as pltpu

X_MIN, Y_MIN, Z_MIN, X_MAX, Y_MAX, Z_MAX = -40.0, -40.0, -3.0, 40.0, 40.0, 1.0
PX, PY = 0.16, 0.16
NX = int(np.round((X_MAX - X_MIN) / PX))
NY = int(np.round((Y_MAX - Y_MIN) / PY))
SENT = NX * NY
MAX_PTS = 100
MAX_PIL = 12000
OUT_CH = 64
EPS = 1e-5

TILE = 96
NTILES = MAX_PIL // TILE


def _prep(p):
    n = p.shape[0]
    x, y, z, w = p[:, 0], p[:, 1], p[:, 2], p[:, 3]
    m = ((x >= X_MIN) & (x < X_MAX) & (y >= Y_MIN) & (y < Y_MAX)
         & (z >= Z_MIN) & (z < Z_MAX))
    xi = jnp.floor((x - X_MIN) / PX).astype(jnp.int32)
    yi = jnp.floor((y - Y_MIN) / PY).astype(jnp.int32)
    key = jnp.where(m, xi * NY + yi, SENT).astype(jnp.int32)
    negz = -z
    iota = jnp.arange(n, dtype=jnp.int32)
    sk, snz, sidx = lax.sort((key, negz, iota), num_keys=2, is_stable=True)
    pts_s = jnp.take(p, sidx, axis=0)

    valid = sk < SENT
    newg = jnp.concatenate([jnp.ones((1,), bool), sk[1:] != sk[:-1]])
    new_valid = newg & valid
    pid = jnp.cumsum(new_valid.astype(jnp.int32)) - 1
    pid_adj = jnp.where(valid, pid, MAX_PIL + 1)
    starts0 = jnp.searchsorted(
        pid_adj, jnp.arange(MAX_PIL + 1, dtype=jnp.int32), side='left')
    P = pid[n - 1] + 1
    P_eff = jnp.minimum(P, MAX_PIL)
    starts = starts0[:MAX_PIL].astype(jnp.int32)
    counts = (starts0[1:] - starts0[:MAX_PIL]).astype(jnp.int32)
    c100 = jnp.minimum(counts, MAX_PTS)
    uk_sorted = jnp.take(sk, starts, axis=0, mode='clip')

    pts_pad = jnp.concatenate(
        [pts_s, jnp.zeros((MAX_PTS, 4), jnp.float32)], axis=0)
    dense = lax.gather(
        pts_pad, starts[:, None],
        lax.GatherDimensionNumbers(offset_dims=(1, 2),
                                   collapsed_slice_dims=(),
                                   start_index_map=(0,)),
        slice_sizes=(MAX_PTS, 4),
        mode=lax.GatherScatterMode.CLIP)
    cm = jnp.arange(MAX_PTS, dtype=jnp.int32)[None, :] < c100[:, None]
    dense = jnp.where(cm[:, :, None], dense, 0.0)

    pm = jnp.arange(MAX_PIL, dtype=jnp.int32) < P_eff
    uk = jnp.where(pm, uk_sorted, 0)
    ux = uk // NY
    uy = uk % NY
    xc = (ux * PX + X_MIN + PX / 2).astype(jnp.float32)
    yc = (uy * PY + Y_MIN + PY / 2).astype(jnp.float32)
    zcol = jnp.zeros((MAX_PIL,), dtype=ux.dtype)
    coords = jnp.stack([zcol, ux, uy], axis=1).astype(jnp.int64)
    cnt = jnp.maximum(P_eff * MAX_PTS, 1).astype(jnp.float32)
    return (dense, pm.astype(jnp.float32),
            xc, yc, coords, cnt)


def _mlp_body(cnt_ref, dense_ref, pm_ref, xc_ref, yc_ref, wm_ref, w2t_ref,
              gb_ref, out_ref, s1, s1q, s2, s2q, ab1, ab2):
    b = pl.program_id(0)
    ph = pl.program_id(1)
    t = pl.program_id(2)
    cnt = cnt_ref[b]

    wm = wm_ref[...]
    dense = dense_ref[0].reshape(TILE * MAX_PTS, 4)
    h1 = jnp.dot(dense, wm[0:4, :], preferred_element_type=jnp.float32)
    h1 = h1.reshape(TILE, MAX_PTS, OUT_CH)
    xc = xc_ref[0]
    yc = yc_ref[0]
    bias = xc * wm[4:5, :] + yc * wm[5:6, :]
    h1 = h1 + bias[:, None, :]
    pm = pm_ref[0]
    pm3 = pm[:, :, None]

    @pl.when(ph == 0)
    def _phase0():
        @pl.when(t == 0)
        def _z0():
            s1[...] = jnp.zeros_like(s1)
            s1q[...] = jnp.zeros_like(s1q)
        h1m = h1 * pm3
        s1[...] += jnp.sum(h1m, axis=(0, 1))[None, :]
        s1q[...] += jnp.sum(h1m * h1m, axis=(0, 1))[None, :]
        out_ref[0] = jnp.zeros((TILE, OUT_CH), jnp.float32)

    @pl.when(ph == 1)
    def _phase1():
        @pl.when(t == 0)
        def _ab1():
            mean = s1[...] / cnt
            var = s1q[...] / cnt - mean * mean
            al = gb_ref[0:1, :] * lax.rsqrt(var + EPS)
            ab1[0:1, :] = al
            ab1[1:2, :] = gb_ref[1:2, :] - mean * al
            s2[...] = jnp.zeros_like(s2)
            s2q[...] = jnp.zeros_like(s2q)
        a1 = jnp.maximum(h1 * ab1[0:1, :][None] + ab1[1:2, :][None], 0.0)
        a1m = (a1 * pm3).reshape(TILE * MAX_PTS, OUT_CH)
        h2m = jnp.dot(a1m, w2t_ref[...], preferred_element_type=jnp.float32)
        s2[...] += jnp.sum(h2m, axis=0)[None, :]
        s2q[...] += jnp.sum(h2m * h2m, axis=0)[None, :]
        out_ref[0] = jnp.zeros((TILE, OUT_CH), jnp.float32)

    @pl.when(ph == 2)
    def _phase2():
        @pl.when(t == 0)
        def _ab2():
            mean = s2[...] / cnt
            var = s2q[...] / cnt - mean * mean
            al = gb_ref[2:3, :] * lax.rsqrt(var + EPS)
            ab2[0:1, :] = al
            ab2[1:2, :] = gb_ref[3:4, :] - mean * al
        a1 = jnp.maximum(h1 * ab1[0:1, :][None] + ab1[1:2, :][None], 0.0)
        h2 = jnp.dot(a1.reshape(TILE * MAX_PTS, OUT_CH), w2t_ref[...],
                     preferred_element_type=jnp.float32)
        a2 = jnp.maximum(h2 * ab2[0:1, :] + ab2[1:2, :], 0.0)
        pooled = jnp.max(a2.reshape(TILE, MAX_PTS, OUT_CH), axis=1)
        out_ref[0] = pooled * pm

    del b, ph, t


def kernel(points, W1, g1, b1, W2, g2, b2):
    B = points.shape[0]
    dense, pm, xc, yc, coords, cnt = jax.vmap(_prep)(points)

    u = W1[:, 4] - W1[:, 6]
    v = W1[:, 5] - W1[:, 7]
    w1eff = jnp.stack([W1[:, 0] + W1[:, 6], W1[:, 1] + W1[:, 7],
                       W1[:, 2], W1[:, 3]], axis=0)
    wm = jnp.concatenate([w1eff, u[None], v[None],
                          jnp.zeros((2, OUT_CH), jnp.float32)], axis=0)
    w2t = W2.T
    gb = jnp.stack([g1, b1, g2, b2], axis=0)

    grid = (B, 3, NTILES)
    feats = pl.pallas_call(
        _mlp_body,
        grid=grid,
        in_specs=[
            pl.BlockSpec(memory_space=pltpu.SMEM),
            pl.BlockSpec((1, TILE, MAX_PTS, 4),
                         lambda b, ph, t: (b, t, 0, 0)),
            pl.BlockSpec((1, TILE, 1), lambda b, ph, t: (b, t, 0)),
            pl.BlockSpec((1, TILE, 1), lambda b, ph, t: (b, t, 0)),
            pl.BlockSpec((1, TILE, 1), lambda b, ph, t: (b, t, 0)),
            pl.BlockSpec((8, OUT_CH), lambda b, ph, t: (0, 0)),
            pl.BlockSpec((OUT_CH, OUT_CH), lambda b, ph, t: (0, 0)),
            pl.BlockSpec((4, OUT_CH), lambda b, ph, t: (0, 0)),
        ],
        out_specs=pl.BlockSpec((1, TILE, OUT_CH), lambda b, ph, t: (b, t, 0)),
        out_shape=jax.ShapeDtypeStruct((B, MAX_PIL, OUT_CH), jnp.float32),
        scratch_shapes=[
            pltpu.VMEM((1, OUT_CH), jnp.float32),
            pltpu.VMEM((1, OUT_CH), jnp.float32),
            pltpu.VMEM((1, OUT_CH), jnp.float32),
            pltpu.VMEM((1, OUT_CH), jnp.float32),
            pltpu.VMEM((2, OUT_CH), jnp.float32),
            pltpu.VMEM((2, OUT_CH), jnp.float32),
        ],
    )(cnt, dense,
      pm[:, :, None], xc[:, :, None], yc[:, :, None], wm, w2t, gb)
    return feats, coords


# speedup vs baseline: 1.4236x; 1.4236x over previous
import jax
import jax.numpy as jnp
import numpy as np
from jax import lax
from jax.experimental import pallas as pl
from jax.experimental.pallas import tpu as pltpu

X_MIN, Y_MIN, Z_MIN, X_MAX, Y_MAX, Z_MAX = -40.0, -40.0, -3.0, 40.0, 40.0, 1.0
PX, PY = 0.16, 0.16
NX = int(np.round((X_MAX - X_MIN) / PX))
NY = int(np.round((Y_MAX - Y_MIN) / PY))
SENT = NX * NY
MAX_PTS = 100
MAX_PIL = 12000
OUT_CH = 64
EPS = 1e-5

TILE = 96
NTILES = MAX_PIL // TILE


def _prep(p):
    n = p.shape[0]
    x, y, z, w = p[:, 0], p[:, 1], p[:, 2], p[:, 3]
    m = ((x >= X_MIN) & (x < X_MAX) & (y >= Y_MIN) & (y < Y_MAX)
         & (z >= Z_MIN) & (z < Z_MAX))
    xi = jnp.floor((x - X_MIN) / PX).astype(jnp.int32)
    yi = jnp.floor((y - Y_MIN) / PY).astype(jnp.int32)
    key = jnp.where(m, xi * NY + yi, SENT).astype(jnp.int32)
    negz = -z
    sk, snz, sx, sy, sw = lax.sort((key, negz, x, y, w), num_keys=2,
                                   is_stable=True)
    pts_s = jnp.stack([sx, sy, -snz, sw], axis=1)

    valid = sk < SENT
    newg = jnp.concatenate([jnp.ones((1,), bool), sk[1:] != sk[:-1]])
    new_valid = newg & valid
    iota = jnp.arange(n, dtype=jnp.int32)
    startkey = jnp.where(new_valid, iota, n).astype(jnp.int32)
    s_sorted, uk_sorted0 = lax.sort((startkey, sk), num_keys=1)

    n_valid = jnp.sum(valid.astype(jnp.int32))
    P = jnp.sum(new_valid.astype(jnp.int32))
    P_eff = jnp.minimum(P, MAX_PIL)
    s_clip = jnp.minimum(s_sorted, n_valid)
    starts = s_clip[:MAX_PIL]
    counts = s_clip[1:MAX_PIL + 1] - starts
    c100 = jnp.minimum(counts, MAX_PTS)
    uk_sorted = uk_sorted0[:MAX_PIL]

    pts_pad = jnp.concatenate(
        [pts_s, jnp.zeros((MAX_PTS, 4), jnp.float32)], axis=0)
    dense = lax.gather(
        pts_pad, starts[:, None],
        lax.GatherDimensionNumbers(offset_dims=(1, 2),
                                   collapsed_slice_dims=(),
                                   start_index_map=(0,)),
        slice_sizes=(MAX_PTS, 4),
        mode=lax.GatherScatterMode.CLIP)
    cm = jnp.arange(MAX_PTS, dtype=jnp.int32)[None, :] < c100[:, None]
    dense = jnp.where(cm[:, :, None], dense, 0.0)

    pm = jnp.arange(MAX_PIL, dtype=jnp.int32) < P_eff
    uk = jnp.where(pm, uk_sorted, 0)
    ux = uk // NY
    uy = uk % NY
    xc = (ux * PX + X_MIN + PX / 2).astype(jnp.float32)
    yc = (uy * PY + Y_MIN + PY / 2).astype(jnp.float32)
    zcol = jnp.zeros((MAX_PIL,), dtype=ux.dtype)
    coords = jnp.stack([zcol, ux, uy], axis=1).astype(jnp.int64)
    cnt = jnp.maximum(P_eff * MAX_PTS, 1).astype(jnp.float32)
    return (dense, pm.astype(jnp.float32),
            xc, yc, coords, cnt)


def _mlp_body(cnt_ref, dense_ref, pm_ref, xc_ref, yc_ref, wm_ref, w2t_ref,
              gb_ref, out_ref, s1, s1q, s2, s2q, ab1, ab2):
    b = pl.program_id(0)
    ph = pl.program_id(1)
    t = pl.program_id(2)
    cnt = cnt_ref[b]

    wm = wm_ref[...]
    dense = dense_ref[0].reshape(TILE * MAX_PTS, 4)
    h1 = jnp.dot(dense, wm[0:4, :], preferred_element_type=jnp.float32)
    h1 = h1.reshape(TILE, MAX_PTS, OUT_CH)
    xc = xc_ref[0]
    yc = yc_ref[0]
    bias = xc * wm[4:5, :] + yc * wm[5:6, :]
    h1 = h1 + bias[:, None, :]
    pm = pm_ref[0]
    pm3 = pm[:, :, None]

    @pl.when(ph == 0)
    def _phase0():
        @pl.when(t == 0)
        def _z0():
            s1[...] = jnp.zeros_like(s1)
            s1q[...] = jnp.zeros_like(s1q)
        h1m = h1 * pm3
        s1[...] += jnp.sum(h1m, axis=(0, 1))[None, :]
        s1q[...] += jnp.sum(h1m * h1m, axis=(0, 1))[None, :]
        out_ref[0] = jnp.zeros((TILE, OUT_CH), jnp.float32)

    @pl.when(ph == 1)
    def _phase1():
        @pl.when(t == 0)
        def _ab1():
            mean = s1[...] / cnt
            var = s1q[...] / cnt - mean * mean
            al = gb_ref[0:1, :] * lax.rsqrt(var + EPS)
            ab1[0:1, :] = al
            ab1[1:2, :] = gb_ref[1:2, :] - mean * al
            s2[...] = jnp.zeros_like(s2)
            s2q[...] = jnp.zeros_like(s2q)
        a1 = jnp.maximum(h1 * ab1[0:1, :][None] + ab1[1:2, :][None], 0.0)
        a1m = (a1 * pm3).reshape(TILE * MAX_PTS, OUT_CH)
        h2m = jnp.dot(a1m, w2t_ref[...], preferred_element_type=jnp.float32)
        s2[...] += jnp.sum(h2m, axis=0)[None, :]
        s2q[...] += jnp.sum(h2m * h2m, axis=0)[None, :]
        out_ref[0] = jnp.zeros((TILE, OUT_CH), jnp.float32)

    @pl.when(ph == 2)
    def _phase2():
        @pl.when(t == 0)
        def _ab2():
            mean = s2[...] / cnt
            var = s2q[...] / cnt - mean * mean
            al = gb_ref[2:3, :] * lax.rsqrt(var + EPS)
            ab2[0:1, :] = al
            ab2[1:2, :] = gb_ref[3:4, :] - mean * al
        a1 = jnp.maximum(h1 * ab1[0:1, :][None] + ab1[1:2, :][None], 0.0)
        h2 = jnp.dot(a1.reshape(TILE * MAX_PTS, OUT_CH), w2t_ref[...],
                     preferred_element_type=jnp.float32)
        a2 = jnp.maximum(h2 * ab2[0:1, :] + ab2[1:2, :], 0.0)
        pooled = jnp.max(a2.reshape(TILE, MAX_PTS, OUT_CH), axis=1)
        out_ref[0] = pooled * pm

    del b, ph, t


def kernel(points, W1, g1, b1, W2, g2, b2):
    B = points.shape[0]
    dense, pm, xc, yc, coords, cnt = jax.vmap(_prep)(points)

    u = W1[:, 4] - W1[:, 6]
    v = W1[:, 5] - W1[:, 7]
    w1eff = jnp.stack([W1[:, 0] + W1[:, 6], W1[:, 1] + W1[:, 7],
                       W1[:, 2], W1[:, 3]], axis=0)
    wm = jnp.concatenate([w1eff, u[None], v[None],
                          jnp.zeros((2, OUT_CH), jnp.float32)], axis=0)
    w2t = W2.T
    gb = jnp.stack([g1, b1, g2, b2], axis=0)

    grid = (B, 3, NTILES)
    feats = pl.pallas_call(
        _mlp_body,
        grid=grid,
        in_specs=[
            pl.BlockSpec(memory_space=pltpu.SMEM),
            pl.BlockSpec((1, TILE, MAX_PTS, 4),
                         lambda b, ph, t: (b, t, 0, 0)),
            pl.BlockSpec((1, TILE, 1), lambda b, ph, t: (b, t, 0)),
            pl.BlockSpec((1, TILE, 1), lambda b, ph, t: (b, t, 0)),
            pl.BlockSpec((1, TILE, 1), lambda b, ph, t: (b, t, 0)),
            pl.BlockSpec((8, OUT_CH), lambda b, ph, t: (0, 0)),
            pl.BlockSpec((OUT_CH, OUT_CH), lambda b, ph, t: (0, 0)),
            pl.BlockSpec((4, OUT_CH), lambda b, ph, t: (0, 0)),
        ],
        out_specs=pl.BlockSpec((1, TILE, OUT_CH), lambda b, ph, t: (b, t, 0)),
        out_shape=jax.ShapeDtypeStruct((B, MAX_PIL, OUT_CH), jnp.float32),
        scratch_shapes=[
            pltpu.VMEM((1, OUT_CH), jnp.float32),
            pltpu.VMEM((1, OUT_CH), jnp.float32),
            pltpu.VMEM((1, OUT_CH), jnp.float32),
            pltpu.VMEM((1, OUT_CH), jnp.float32),
            pltpu.VMEM((2, OUT_CH), jnp.float32),
            pltpu.VMEM((2, OUT_CH), jnp.float32),
        ],
    )(cnt, dense,
      pm[:, :, None], xc[:, :, None], yc[:, :, None], wm, w2t, gb)
    return feats, coords
